# baseline (speedup 1.0000x reference)
import functools

import jax
import jax.numpy as jnp
from jax import lax
from jax.experimental import pallas as pl
from jax.experimental.pallas import tpu as pltpu
from jax.experimental.pallas import tpu_sc as plsc

NC = 2
NS = 16
NW = NC * NS
CH = 128


def _sc_aggregate(x, idx_all, n_pad):
    n, d = x.shape
    nwin = idx_all.shape[0]
    wpw = nwin // NW
    na = n + n_pad
    nca = na // CH
    nrc = n // CH
    nt = n - nrc * CH

    mesh = plsc.VectorSubcoreMesh(core_axis_name="c", subcore_axis_name="s")

    scratch = [
        pltpu.VMEM((2, CH), jnp.int32),
        pltpu.VMEM((2, CH), jnp.int32),
        pltpu.VMEM((2, CH), jnp.int32),
        pltpu.VMEM((2, CH), jnp.int32),
        pltpu.VMEM((CH, d), jnp.float32),
        pltpu.VMEM((CH, d), jnp.float32),
        pltpu.VMEM_SHARED((na, d), jnp.float32),
        pltpu.SemaphoreType.DMA,
        pltpu.SemaphoreType.DMA,
        pltpu.SemaphoreType.DMA,
        pltpu.SemaphoreType.DMA,
        pltpu.SemaphoreType.DMA,
        pltpu.SemaphoreType.DMA,
        pltpu.SemaphoreType.DMA,
        pltpu.SemaphoreType.DMA,
    ]

    @functools.partial(
        pl.kernel,
        out_type=jax.ShapeDtypeStruct((NC, n, d), jnp.float32),
        mesh=mesh,
        scratch_types=scratch,
    )
    def agg_kernel(x_hbm, idx_hbm, out_hbm, ib0, ib1, ib2, ib3, rows0,
                   rows1, agg_sh, is0, is1, is2, is3, gs0, gs1, ss0, ss1):
        cid = lax.axis_index("c")
        sid = lax.axis_index("s")
        wid = sid * NC + cid
        wbase = wid * wpw
        ibufs = (ib0, ib1, ib2, ib3)
        isems = (is0, is1, is2, is3)
        rbufs = (rows0, rows1)
        gsems = (gs0, gs1)
        ssems = (ss0, ss1)

        def idx_req(j, t):
            pltpu.async_copy(idx_hbm.at[wbase + j], ibufs[t], isems[t])

        def idx_wait(t):
            pltpu.make_async_copy(idx_hbm.at[0], ibufs[t], isems[t]).wait()

        def gather(t_idx, t_row):
            pltpu.async_copy(x_hbm.at[ibufs[t_idx].at[0]], rbufs[t_row],
                             gsems[t_row])

        def rows_wait(t_row):
            pltpu.make_async_copy(x_hbm.at[pl.ds(0, CH)], rbufs[t_row],
                                  gsems[t_row]).wait()

        def scat(t_idx, t_row):
            pltpu.async_copy(rbufs[t_row], agg_sh.at[ibufs[t_idx].at[1]],
                             ssems[t_row], add=True)

        def scat_wait(t_row):
            pltpu.make_async_copy(x_hbm.at[pl.ds(0, CH)], rbufs[t_row],
                                  ssems[t_row]).wait()

        for j in range(3):
            idx_req(j, j)

        zv = jnp.zeros((16,), jnp.float32)

        def zrow(i, carry):
            for t in range(d // 16):
                rows0[i, pl.ds(t * 16, 16)] = zv
            return carry

        lax.fori_loop(0, CH, zrow, 0)
        for k in range(nca // NS):
            c = sid * (nca // NS) + k
            r0 = pl.multiple_of(c * CH, 8)
            pltpu.sync_copy(rows0, agg_sh.at[pl.ds(r0, CH)])
        plsc.subcore_barrier()

        idx_wait(0)
        gather(0, 0)

        def body(k, carry):
            for t in range(4):
                j = 4 * k + t
                r = t % 2
                rows_wait(r)
                scat(t, r)

                @pl.when(j >= 1)
                def _():
                    scat_wait(1 - r)

                @pl.when(j + 1 < wpw)
                def _():
                    idx_wait((t + 1) % 4)
                    gather((t + 1) % 4, 1 - r)

                @pl.when(j + 3 < wpw)
                def _():
                    idx_req(j + 3, (t + 3) % 4)
            return carry

        lax.fori_loop(0, wpw // 4, body, 0)
        scat_wait((wpw - 1) % 2)
        plsc.subcore_barrier()

        for k in range((nrc + NS - 1) // NS):
            c = sid + NS * k

            @pl.when(c < nrc)
            def _():
                r0 = pl.multiple_of(c * CH, 8)
                pltpu.sync_copy(agg_sh.at[pl.ds(r0, CH)], rows0)
                pltpu.sync_copy(rows0, out_hbm.at[cid, pl.ds(r0, CH)])

        if nt:

            @pl.when(sid == 0)
            def _():
                r0 = nrc * CH
                pltpu.sync_copy(agg_sh.at[pl.ds(r0, nt)],
                                rows1.at[pl.ds(0, nt)])
                pltpu.sync_copy(rows1.at[pl.ds(0, nt)],
                                out_hbm.at[cid, pl.ds(r0, nt)])

    return agg_kernel(x, idx_all)


def _tc_mlp(x, partials, w, b2d):
    n, d = x.shape
    br = 1000

    def body(x_ref, p_ref, w_ref, b_ref, o_ref):
        h = x_ref[...] + p_ref[0] + p_ref[1]
        o_ref[...] = (
            jnp.dot(h, w_ref[...], preferred_element_type=jnp.float32)
            + b_ref[...]
        )

    return pl.pallas_call(
        body,
        grid=(n // br,),
        in_specs=[
            pl.BlockSpec((br, d), lambda i: (i, 0)),
            pl.BlockSpec((NC, br, d), lambda i: (0, i, 0)),
            pl.BlockSpec((d, d), lambda i: (0, 0)),
            pl.BlockSpec((1, d), lambda i: (0, 0)),
        ],
        out_specs=pl.BlockSpec((br, d), lambda i: (i, 0)),
        out_shape=jax.ShapeDtypeStruct((n, d), jnp.float32),
    )(x, partials, w, b2d)


def kernel(X, edge_index, W, b):
    n, d = X.shape
    e = edge_index.shape[1]
    n_pad_rows = -(-(n + 240) // CH) * CH - n
    wpw = -(-e // (NW * CH * 4)) * 4
    e_pad = wpw * NW * CH - e
    pad_ar = jnp.arange(e_pad, dtype=jnp.int32)
    src = jnp.concatenate([edge_index[0], pad_ar % n]).reshape(-1, CH)
    dst = jnp.concatenate(
        [edge_index[1], n + pad_ar % n_pad_rows]).reshape(-1, CH)
    idx_all = jnp.stack([src, dst], axis=1)
    partials = _sc_aggregate(X, idx_all, n_pad_rows)
    return _tc_mlp(X, partials, W, b.reshape(1, d))


# speedup vs baseline: 1.3026x; 1.3026x over previous
import functools

import jax
import jax.numpy as jnp
from jax import lax
from jax.experimental import pallas as pl
from jax.experimental.pallas import tpu as pltpu
from jax.experimental.pallas import tpu_sc as plsc

NC = 2
NS = 16
NW = NC * NS
CH = 128


def _sc_aggregate(x, edge_index):
    n, d = x.shape
    e = edge_index.shape[1]
    nwin = e // CH
    nw_w = nwin // NW
    extra = nwin - nw_w * NW
    nmain = (nw_w // 4) * 4
    nrc = n // CH
    nt = n - nrc * CH

    mesh = plsc.VectorSubcoreMesh(core_axis_name="c", subcore_axis_name="s")

    scratch = [
        pltpu.VMEM((CH,), jnp.int32),
        pltpu.VMEM((CH,), jnp.int32),
        pltpu.VMEM((CH,), jnp.int32),
        pltpu.VMEM((CH,), jnp.int32),
        pltpu.VMEM((CH,), jnp.int32),
        pltpu.VMEM((CH,), jnp.int32),
        pltpu.VMEM((CH,), jnp.int32),
        pltpu.VMEM((CH,), jnp.int32),
        pltpu.VMEM((CH, d), jnp.float32),
        pltpu.VMEM((CH, d), jnp.float32),
        pltpu.VMEM_SHARED((n, d), jnp.float32),
        pltpu.SemaphoreType.DMA,
        pltpu.SemaphoreType.DMA,
        pltpu.SemaphoreType.DMA,
        pltpu.SemaphoreType.DMA,
        pltpu.SemaphoreType.DMA,
        pltpu.SemaphoreType.DMA,
    ]

    @functools.partial(
        pl.kernel,
        out_type=jax.ShapeDtypeStruct((NC, n, d), jnp.float32),
        mesh=mesh,
        scratch_types=scratch,
    )
    def agg_kernel(x_hbm, edge_hbm, out_hbm, sb0, sb1, sb2, sb3, db0, db1,
                   db2, db3, rows0, rows1, agg_sh, is0, is1, is2, is3,
                   gs0, gs1):
        cid = lax.axis_index("c")
        sid = lax.axis_index("s")
        wid = sid * NC + cid
        wbase = wid * nw_w
        sbufs = (sb0, sb1, sb2, sb3)
        dbufs = (db0, db1, db2, db3)
        isems = (is0, is1, is2, is3)
        rbufs = (rows0, rows1)
        gsems = (gs0, gs1)

        def idx_req(j, t):
            off = pl.multiple_of((wbase + j) * CH, 8)
            pltpu.async_copy(edge_hbm.at[0, pl.ds(off, CH)], sbufs[t],
                             isems[t])
            pltpu.async_copy(edge_hbm.at[1, pl.ds(off, CH)], dbufs[t],
                             isems[t])

        def idx_wait(t):
            pltpu.make_async_copy(edge_hbm.at[0, pl.ds(0, CH)], sbufs[t],
                                  isems[t]).wait()
            pltpu.make_async_copy(edge_hbm.at[0, pl.ds(0, CH)], dbufs[t],
                                  isems[t]).wait()

        def gather(t_idx, t_row):
            pltpu.async_copy(x_hbm.at[sbufs[t_idx]], rbufs[t_row],
                             gsems[t_row])

        def rows_wait(t_row):
            pltpu.make_async_copy(x_hbm.at[pl.ds(0, CH)], rbufs[t_row],
                                  gsems[t_row]).wait()

        def scat(t_idx, t_row):
            pltpu.sync_copy(rbufs[t_row], agg_sh.at[dbufs[t_idx]],
                            add=True)

        for j in range(3):
            idx_req(j, j)

        zv = jnp.zeros((16,), jnp.float32)

        def zrow(i, carry):
            for t in range(d // 16):
                rows0[i, pl.ds(t * 16, 16)] = zv
            return carry

        lax.fori_loop(0, CH, zrow, 0)
        for k in range((nrc + NS - 1) // NS):
            c = sid + NS * k

            @pl.when(c < nrc)
            def _():
                r0 = pl.multiple_of(c * CH, 8)
                pltpu.sync_copy(rows0, agg_sh.at[pl.ds(r0, CH)])

        if nt:

            @pl.when(sid == 0)
            def _():
                pltpu.sync_copy(rows0.at[pl.ds(0, nt)],
                                agg_sh.at[pl.ds(nrc * CH, nt)])

        plsc.subcore_barrier()

        idx_wait(0)
        gather(0, 0)

        def body(k, carry):
            for t in range(4):
                j = 4 * k + t

                @pl.when(j + 1 < nw_w)
                def _():
                    idx_wait((t + 1) % 4)
                    gather((t + 1) % 4, (t + 1) % 2)

                rows_wait(t % 2)
                scat(t, t % 2)

                @pl.when(j + 3 < nw_w)
                def _():
                    idx_req(j + 3, (t + 3) % 4)
            return carry

        lax.fori_loop(0, nmain // 4, body, 0)

        for je in range(nmain, nw_w):
            t = je % 4
            if je + 1 < nw_w:
                idx_wait((t + 1) % 4)
                gather((t + 1) % 4, (t + 1) % 2)
            rows_wait(t % 2)
            scat(t, t % 2)

        if extra:

            @pl.when(wid < extra)
            def _():
                off = pl.multiple_of((NW * nw_w + wid) * CH, 8)
                pltpu.sync_copy(edge_hbm.at[0, pl.ds(off, CH)], sbufs[0])
                pltpu.sync_copy(edge_hbm.at[1, pl.ds(off, CH)], dbufs[0])
                gather(0, 0)
                rows_wait(0)
                scat(0, 0)

        plsc.subcore_barrier()

        for k in range((nrc + NS - 1) // NS):
            c = sid + NS * k

            @pl.when(c < nrc)
            def _():
                r0 = pl.multiple_of(c * CH, 8)
                pltpu.sync_copy(agg_sh.at[pl.ds(r0, CH)],
                                out_hbm.at[cid, pl.ds(r0, CH)])

        if nt:

            @pl.when(sid == 0)
            def _():
                r0 = nrc * CH
                pltpu.sync_copy(agg_sh.at[pl.ds(r0, nt)],
                                out_hbm.at[cid, pl.ds(r0, nt)])

    return agg_kernel(x, edge_index)


def _tc_mlp(x, partials, w, b2d):
    n, d = x.shape
    br = 1000

    def body(x_ref, p_ref, w_ref, b_ref, o_ref):
        h = x_ref[...] + p_ref[0] + p_ref[1]
        o_ref[...] = (
            jnp.dot(h, w_ref[...], preferred_element_type=jnp.float32)
            + b_ref[...]
        )

    return pl.pallas_call(
        body,
        grid=(n // br,),
        in_specs=[
            pl.BlockSpec((br, d), lambda i: (i, 0)),
            pl.BlockSpec((NC, br, d), lambda i: (0, i, 0)),
            pl.BlockSpec((d, d), lambda i: (0, 0)),
            pl.BlockSpec((1, d), lambda i: (0, 0)),
        ],
        out_specs=pl.BlockSpec((br, d), lambda i: (i, 0)),
        out_shape=jax.ShapeDtypeStruct((n, d), jnp.float32),
    )(x, partials, w, b2d)


def kernel(X, edge_index, W, b):
    n, d = X.shape
    partials = _sc_aggregate(X, edge_index)
    return _tc_mlp(X, partials, W, b.reshape(1, d))


# speedup vs baseline: 1.4089x; 1.0816x over previous
import functools

import jax
import jax.numpy as jnp
from jax import lax
from jax.experimental import pallas as pl
from jax.experimental.pallas import tpu as pltpu
from jax.experimental.pallas import tpu_sc as plsc

NC = 2
NS = 16
NW = NC * NS
CH = 128


def _sc_aggregate(x, edge_index):
    n, d = x.shape
    e = edge_index.shape[1]
    nwin = e // CH
    nw_w = nwin // NW
    extra = nwin - nw_w * NW
    nmain = (nw_w // 4) * 4
    nrc = n // CH
    nt = n - nrc * CH

    mesh = plsc.VectorSubcoreMesh(core_axis_name="c", subcore_axis_name="s")

    scratch = [
        pltpu.VMEM((CH,), jnp.int32),
        pltpu.VMEM((CH,), jnp.int32),
        pltpu.VMEM((CH,), jnp.int32),
        pltpu.VMEM((CH,), jnp.int32),
        pltpu.VMEM((CH,), jnp.int32),
        pltpu.VMEM((CH,), jnp.int32),
        pltpu.VMEM((CH,), jnp.int32),
        pltpu.VMEM((CH,), jnp.int32),
        pltpu.VMEM((CH, d), jnp.float32),
        pltpu.VMEM((CH, d), jnp.float32),
        pltpu.VMEM((CH, d), jnp.float32),
        pltpu.VMEM_SHARED((n, d), jnp.float32),
        pltpu.SemaphoreType.DMA,
        pltpu.SemaphoreType.DMA,
        pltpu.SemaphoreType.DMA,
        pltpu.SemaphoreType.DMA,
        pltpu.SemaphoreType.DMA,
        pltpu.SemaphoreType.DMA,
        pltpu.SemaphoreType.DMA,
    ]

    @functools.partial(
        pl.kernel,
        out_type=jax.ShapeDtypeStruct((NC, n, d), jnp.float32),
        mesh=mesh,
        scratch_types=scratch,
    )
    def agg_kernel(x_hbm, edge_hbm, out_hbm, sb0, sb1, sb2, sb3, db0, db1,
                   db2, db3, rows0, rows1, rows2, agg_sh, is0, is1, is2,
                   is3, gs0, gs1, gs2):
        cid = lax.axis_index("c")
        sid = lax.axis_index("s")
        wid = sid * NC + cid
        wbase = wid * nw_w
        sbufs = (sb0, sb1, sb2, sb3)
        dbufs = (db0, db1, db2, db3)
        isems = (is0, is1, is2, is3)
        rbufs = (rows0, rows1, rows2)
        gsems = (gs0, gs1, gs2)

        def idx_req(j, t):
            off = pl.multiple_of((wbase + j) * CH, 8)
            pltpu.async_copy(edge_hbm.at[0, pl.ds(off, CH)], sbufs[t],
                             isems[t])
            pltpu.async_copy(edge_hbm.at[1, pl.ds(off, CH)], dbufs[t],
                             isems[t])

        def idx_wait(t):
            pltpu.make_async_copy(edge_hbm.at[0, pl.ds(0, CH)], sbufs[t],
                                  isems[t]).wait()
            pltpu.make_async_copy(edge_hbm.at[0, pl.ds(0, CH)], dbufs[t],
                                  isems[t]).wait()

        def gather(t_idx, t_row):
            pltpu.async_copy(x_hbm.at[sbufs[t_idx]], rbufs[t_row],
                             gsems[t_row])

        def rows_wait(t_row):
            pltpu.make_async_copy(x_hbm.at[pl.ds(0, CH)], rbufs[t_row],
                                  gsems[t_row]).wait()

        def scat(t_idx, t_row):
            pltpu.sync_copy(rbufs[t_row], agg_sh.at[dbufs[t_idx]],
                            add=True)

        for j in range(3):
            idx_req(j, j)

        zv = jnp.zeros((16,), jnp.float32)

        def zrow(i, carry):
            for t in range(d // 16):
                rows0[i, pl.ds(t * 16, 16)] = zv
            return carry

        lax.fori_loop(0, CH, zrow, 0)
        for k in range((nrc + NS - 1) // NS):
            c = sid + NS * k

            @pl.when(c < nrc)
            def _():
                r0 = pl.multiple_of(c * CH, 8)
                pltpu.sync_copy(rows0, agg_sh.at[pl.ds(r0, CH)])

        if nt:

            @pl.when(sid == 0)
            def _():
                pltpu.sync_copy(rows0.at[pl.ds(0, nt)],
                                agg_sh.at[pl.ds(nrc * CH, nt)])

        plsc.subcore_barrier()

        nfull = nw_w // 12
        ep = nw_w - 12 * nfull
        if ep < 3 and nfull > 0:
            nfull -= 1
            ep += 12

        idx_wait(0)
        gather(0, 0)
        if nw_w > 1:
            idx_wait(1)
            gather(1, 1)

        def body(k, carry):
            for u in range(12):
                j = 12 * k + u
                idx_wait((u + 2) % 4)
                gather((u + 2) % 4, (u + 2) % 3)
                idx_req(j + 3, (u + 3) % 4)
                rows_wait(u % 3)
                scat(u % 4, u % 3)
            return carry

        if nfull:
            lax.fori_loop(0, nfull, body, 0)

        for je in range(12 * nfull, nw_w):
            if je + 2 < nw_w:
                idx_wait((je + 2) % 4)
                gather((je + 2) % 4, (je + 2) % 3)
            if je + 3 < nw_w:
                idx_req(je + 3, (je + 3) % 4)
            rows_wait(je % 3)
            scat(je % 4, je % 3)

        if extra:

            @pl.when(wid < extra)
            def _():
                off = pl.multiple_of((NW * nw_w + wid) * CH, 8)
                pltpu.sync_copy(edge_hbm.at[0, pl.ds(off, CH)], sbufs[0])
                pltpu.sync_copy(edge_hbm.at[1, pl.ds(off, CH)], dbufs[0])
                gather(0, 0)
                rows_wait(0)
                scat(0, 0)

        plsc.subcore_barrier()

        for k in range((nrc + NS - 1) // NS):
            c = sid + NS * k

            @pl.when(c < nrc)
            def _():
                r0 = pl.multiple_of(c * CH, 8)
                pltpu.sync_copy(agg_sh.at[pl.ds(r0, CH)],
                                out_hbm.at[cid, pl.ds(r0, CH)])

        if nt:

            @pl.when(sid == 0)
            def _():
                r0 = nrc * CH
                pltpu.sync_copy(agg_sh.at[pl.ds(r0, nt)],
                                out_hbm.at[cid, pl.ds(r0, nt)])

    return agg_kernel(x, edge_index)


def _tc_mlp(x, partials, w, b2d):
    n, d = x.shape
    br = 1000

    def body(x_ref, p_ref, w_ref, b_ref, o_ref):
        h = x_ref[...] + p_ref[0] + p_ref[1]
        o_ref[...] = (
            jnp.dot(h, w_ref[...], preferred_element_type=jnp.float32)
            + b_ref[...]
        )

    return pl.pallas_call(
        body,
        grid=(n // br,),
        in_specs=[
            pl.BlockSpec((br, d), lambda i: (i, 0)),
            pl.BlockSpec((NC, br, d), lambda i: (0, i, 0)),
            pl.BlockSpec((d, d), lambda i: (0, 0)),
            pl.BlockSpec((1, d), lambda i: (0, 0)),
        ],
        out_specs=pl.BlockSpec((br, d), lambda i: (i, 0)),
        out_shape=jax.ShapeDtypeStruct((n, d), jnp.float32),
    )(x, partials, w, b2d)


def kernel(X, edge_index, W, b):
    n, d = X.shape
    partials = _sc_aggregate(X, edge_index)
    return _tc_mlp(X, partials, W, b.reshape(1, d))


# speedup vs baseline: 1.4221x; 1.0094x over previous
import functools

import jax
import jax.numpy as jnp
from jax import lax
from jax.experimental import pallas as pl
from jax.experimental.pallas import tpu as pltpu
from jax.experimental.pallas import tpu_sc as plsc

NC = 2
NS = 16
NW = NC * NS
CH = 128


def _sc_aggregate(x, idx2d):
    n, d = x.shape
    nwin = idx2d.shape[0]
    nw_w = nwin // NW
    extra = nwin - nw_w * NW
    nmain = (nw_w // 4) * 4
    nrc = n // CH
    nt = n - nrc * CH

    mesh = plsc.VectorSubcoreMesh(core_axis_name="c", subcore_axis_name="s")

    scratch = [
        pltpu.VMEM((2, CH), jnp.int32),
        pltpu.VMEM((2, CH), jnp.int32),
        pltpu.VMEM((2, CH), jnp.int32),
        pltpu.VMEM((2, CH), jnp.int32),
        pltpu.VMEM((CH, d), jnp.float32),
        pltpu.VMEM((CH, d), jnp.float32),
        pltpu.VMEM((CH, d), jnp.float32),
        pltpu.VMEM_SHARED((n, d), jnp.float32),
        pltpu.SemaphoreType.DMA,
        pltpu.SemaphoreType.DMA,
        pltpu.SemaphoreType.DMA,
        pltpu.SemaphoreType.DMA,
        pltpu.SemaphoreType.DMA,
        pltpu.SemaphoreType.DMA,
        pltpu.SemaphoreType.DMA,
    ]

    @functools.partial(
        pl.kernel,
        out_type=jax.ShapeDtypeStruct((NC, n, d), jnp.float32),
        mesh=mesh,
        scratch_types=scratch,
    )
    def agg_kernel(x_hbm, idx_hbm, out_hbm, ib0, ib1, ib2, ib3, rows0,
                   rows1, rows2, agg_sh, is0, is1, is2, is3, gs0, gs1,
                   gs2):
        cid = lax.axis_index("c")
        sid = lax.axis_index("s")
        wid = sid * NC + cid
        wbase = wid * nw_w
        ibufs = (ib0, ib1, ib2, ib3)
        isems = (is0, is1, is2, is3)
        rbufs = (rows0, rows1, rows2)
        gsems = (gs0, gs1, gs2)

        def idx_req(j, t):
            pltpu.async_copy(idx_hbm.at[wbase + j], ibufs[t], isems[t])

        def idx_wait(t):
            pltpu.make_async_copy(idx_hbm.at[0], ibufs[t], isems[t]).wait()

        def gather(t_idx, t_row):
            pltpu.async_copy(x_hbm.at[ibufs[t_idx].at[0]], rbufs[t_row],
                             gsems[t_row])

        def rows_wait(t_row):
            pltpu.make_async_copy(x_hbm.at[pl.ds(0, CH)], rbufs[t_row],
                                  gsems[t_row]).wait()

        def scat(t_idx, t_row):
            pltpu.sync_copy(rbufs[t_row], agg_sh.at[ibufs[t_idx].at[1]],
                            add=True)

        for j in range(3):
            idx_req(j, j)

        zv = jnp.zeros((16,), jnp.float32)

        def zrow(i, carry):
            for t in range(d // 16):
                rows0[i, pl.ds(t * 16, 16)] = zv
            return carry

        lax.fori_loop(0, CH, zrow, 0)
        for k in range((nrc + NS - 1) // NS):
            c = sid + NS * k

            @pl.when(c < nrc)
            def _():
                r0 = pl.multiple_of(c * CH, 8)
                pltpu.sync_copy(rows0, agg_sh.at[pl.ds(r0, CH)])

        if nt:

            @pl.when(sid == 0)
            def _():
                pltpu.sync_copy(rows0.at[pl.ds(0, nt)],
                                agg_sh.at[pl.ds(nrc * CH, nt)])

        plsc.subcore_barrier()

        nfull = nw_w // 12
        ep = nw_w - 12 * nfull
        if ep < 3 and nfull > 0:
            nfull -= 1
            ep += 12

        idx_wait(0)
        gather(0, 0)
        if nw_w > 1:
            idx_wait(1)
            gather(1, 1)

        def body(k, carry):
            for u in range(12):
                j = 12 * k + u
                idx_wait((u + 2) % 4)
                gather((u + 2) % 4, (u + 2) % 3)
                idx_req(j + 3, (u + 3) % 4)
                rows_wait(u % 3)
                scat(u % 4, u % 3)
            return carry

        if nfull:
            lax.fori_loop(0, nfull, body, 0)

        for je in range(12 * nfull, nw_w):
            if je + 2 < nw_w:
                idx_wait((je + 2) % 4)
                gather((je + 2) % 4, (je + 2) % 3)
            if je + 3 < nw_w:
                idx_req(je + 3, (je + 3) % 4)
            rows_wait(je % 3)
            scat(je % 4, je % 3)

        if extra:

            @pl.when(wid < extra)
            def _():
                pltpu.sync_copy(idx_hbm.at[NW * nw_w + wid], ibufs[0])
                gather(0, 0)
                rows_wait(0)
                scat(0, 0)

        plsc.subcore_barrier()

        for k in range((nrc + NS - 1) // NS):
            c = sid + NS * k

            @pl.when(c < nrc)
            def _():
                r0 = pl.multiple_of(c * CH, 8)
                pltpu.sync_copy(agg_sh.at[pl.ds(r0, CH)],
                                out_hbm.at[cid, pl.ds(r0, CH)])

        if nt:

            @pl.when(sid == 0)
            def _():
                r0 = nrc * CH
                pltpu.sync_copy(agg_sh.at[pl.ds(r0, nt)],
                                out_hbm.at[cid, pl.ds(r0, nt)])

    return agg_kernel(x, idx2d)


def _tc_mlp(x, partials, w, b2d):
    n, d = x.shape
    br = 1000

    def body(x_ref, p_ref, w_ref, b_ref, o_ref):
        h = x_ref[...] + p_ref[0] + p_ref[1]
        o_ref[...] = (
            jnp.dot(h, w_ref[...], preferred_element_type=jnp.float32)
            + b_ref[...]
        )

    return pl.pallas_call(
        body,
        grid=(n // br,),
        in_specs=[
            pl.BlockSpec((br, d), lambda i: (i, 0)),
            pl.BlockSpec((NC, br, d), lambda i: (0, i, 0)),
            pl.BlockSpec((d, d), lambda i: (0, 0)),
            pl.BlockSpec((1, d), lambda i: (0, 0)),
        ],
        out_specs=pl.BlockSpec((br, d), lambda i: (i, 0)),
        out_shape=jax.ShapeDtypeStruct((n, d), jnp.float32),
    )(x, partials, w, b2d)


def kernel(X, edge_index, W, b):
    n, d = X.shape
    idx2d = jnp.swapaxes(edge_index.reshape(2, -1, CH), 0, 1)
    partials = _sc_aggregate(X, idx2d)
    return _tc_mlp(X, partials, W, b.reshape(1, d))


# speedup vs baseline: 1.4553x; 1.0233x over previous
import functools

import jax
import jax.numpy as jnp
from jax import lax
from jax.experimental import pallas as pl
from jax.experimental.pallas import tpu as pltpu
from jax.experimental.pallas import tpu_sc as plsc

NC = 2
NS = 16
NW = NC * NS
CH = 128


def _sc_aggregate(x, idx2d):
    n, d = x.shape
    nwin = idx2d.shape[0]
    nw_w = nwin // NW
    extra = nwin - nw_w * NW
    nmain = (nw_w // 4) * 4
    nrc = n // CH
    nt = n - nrc * CH

    mesh = plsc.VectorSubcoreMesh(core_axis_name="c", subcore_axis_name="s")

    scratch = [
        pltpu.VMEM((2, CH), jnp.int32),
        pltpu.VMEM((2, CH), jnp.int32),
        pltpu.VMEM((2, CH), jnp.int32),
        pltpu.VMEM((2, CH), jnp.int32),
        pltpu.VMEM((CH, d), jnp.float32),
        pltpu.VMEM((CH, d), jnp.float32),
        pltpu.VMEM((CH, d), jnp.float32),
        pltpu.VMEM_SHARED((n, d), jnp.float32),
        pltpu.SemaphoreType.DMA,
        pltpu.SemaphoreType.DMA,
        pltpu.SemaphoreType.DMA,
        pltpu.SemaphoreType.DMA,
        pltpu.SemaphoreType.DMA,
        pltpu.SemaphoreType.DMA,
        pltpu.SemaphoreType.DMA,
    ]

    @functools.partial(
        pl.kernel,
        out_type=jax.ShapeDtypeStruct((NC, n, d), jnp.float32),
        mesh=mesh,
        scratch_types=scratch,
    )
    def agg_kernel(x_hbm, idx_hbm, out_hbm, ib0, ib1, ib2, ib3, rows0,
                   rows1, rows2, agg_sh, is0, is1, is2, is3, gs0, gs1,
                   gs2):
        cid = lax.axis_index("c")
        sid = lax.axis_index("s")
        wid = sid * NC + cid
        wbase = wid * nw_w
        ibufs = (ib0, ib1, ib2, ib3)
        isems = (is0, is1, is2, is3)
        rbufs = (rows0, rows1, rows2)
        gsems = (gs0, gs1, gs2)

        def idx_req(j, t):
            pltpu.async_copy(idx_hbm.at[wbase + j], ibufs[t], isems[t])

        def idx_wait(t):
            pltpu.make_async_copy(idx_hbm.at[0], ibufs[t], isems[t]).wait()

        def gather(t_idx, t_row):
            pltpu.async_copy(x_hbm.at[ibufs[t_idx].at[0]], rbufs[t_row],
                             gsems[t_row])

        def rows_wait(t_row):
            pltpu.make_async_copy(x_hbm.at[pl.ds(0, CH)], rbufs[t_row],
                                  gsems[t_row]).wait()

        def scat(t_idx, t_row):
            pltpu.sync_copy(rbufs[t_row], agg_sh.at[ibufs[t_idx].at[1]],
                            add=True)

        for j in range(3):
            idx_req(j, j)

        zv = jnp.zeros((16,), jnp.float32)

        def zrow(i, carry):
            for t in range(d // 16):
                rows0[i, pl.ds(t * 16, 16)] = zv
            return carry

        lax.fori_loop(0, CH, zrow, 0)
        for k in range((nrc + NS - 1) // NS):
            c = sid + NS * k

            @pl.when(c < nrc)
            def _():
                r0 = pl.multiple_of(c * CH, 8)
                pltpu.sync_copy(rows0, agg_sh.at[pl.ds(r0, CH)])

        if nt:

            @pl.when(sid == 0)
            def _():
                pltpu.sync_copy(rows0.at[pl.ds(0, nt)],
                                agg_sh.at[pl.ds(nrc * CH, nt)])

        plsc.subcore_barrier()

        nfull = nw_w // 12
        ep = nw_w - 12 * nfull
        if ep < 3 and nfull > 0:
            nfull -= 1
            ep += 12

        idx_wait(0)
        gather(0, 0)
        if nw_w > 1:
            idx_wait(1)
            gather(1, 1)

        def body(k, carry):
            for u in range(12):
                j = 12 * k + u
                idx_wait((u + 2) % 4)
                gather((u + 2) % 4, (u + 2) % 3)
                idx_req(j + 3, (u + 3) % 4)
                rows_wait(u % 3)
                scat(u % 4, u % 3)
            return carry

        if nfull:
            lax.fori_loop(0, nfull, body, 0)

        for je in range(12 * nfull, nw_w):
            if je + 2 < nw_w:
                idx_wait((je + 2) % 4)
                gather((je + 2) % 4, (je + 2) % 3)
            if je + 3 < nw_w:
                idx_req(je + 3, (je + 3) % 4)
            rows_wait(je % 3)
            scat(je % 4, je % 3)

        if extra:

            @pl.when(wid < extra)
            def _():
                pltpu.sync_copy(idx_hbm.at[NW * nw_w + wid], ibufs[0])
                gather(0, 0)
                rows_wait(0)
                scat(0, 0)

        plsc.subcore_barrier()

        for k in range((nrc + NS - 1) // NS):
            c = sid + NS * k

            @pl.when(c < nrc)
            def _():
                r0 = pl.multiple_of(c * CH, 8)
                pltpu.sync_copy(agg_sh.at[pl.ds(r0, CH)],
                                out_hbm.at[cid, pl.ds(r0, CH)])

        if nt:

            @pl.when(sid == 0)
            def _():
                r0 = nrc * CH
                pltpu.sync_copy(agg_sh.at[pl.ds(r0, nt)],
                                out_hbm.at[cid, pl.ds(r0, nt)])

    return agg_kernel(x, idx2d)


def _tc_mlp(x, partials, w, b2d):
    n, d = x.shape
    br = 2000

    def body(x_ref, p_ref, w_ref, b_ref, o_ref):
        h = x_ref[...] + p_ref[0] + p_ref[1]
        o_ref[...] = (
            jnp.dot(h, w_ref[...], preferred_element_type=jnp.float32)
            + b_ref[...]
        )

    return pl.pallas_call(
        body,
        grid=(n // br,),
        in_specs=[
            pl.BlockSpec((br, d), lambda i: (i, 0)),
            pl.BlockSpec((NC, br, d), lambda i: (0, i, 0)),
            pl.BlockSpec((d, d), lambda i: (0, 0)),
            pl.BlockSpec((1, d), lambda i: (0, 0)),
        ],
        out_specs=pl.BlockSpec((br, d), lambda i: (i, 0)),
        out_shape=jax.ShapeDtypeStruct((n, d), jnp.float32),
    )(x, partials, w, b2d)


def kernel(X, edge_index, W, b):
    n, d = X.shape
    idx2d = jnp.swapaxes(edge_index.reshape(2, -1, CH), 0, 1)
    partials = _sc_aggregate(X, idx2d)
    return _tc_mlp(X, partials, W, b.reshape(1, d))
